# baseline (speedup 1.0000x reference)
import functools

import jax
import jax.numpy as jnp
from jax import lax
from jax.experimental import pallas as pl
from jax.experimental.pallas import tpu as pltpu
from jax.experimental.pallas import tpu_sc as plsc

NUM_CORES = 2
NUM_SUBCORES = 16
NUM_WORKERS = NUM_CORES * NUM_SUBCORES
LANES = 16
TILE = 128
CHUNK = 512
IDX_CHUNK = 2048
ROWCAP = 128


@functools.lru_cache(maxsize=None)
def _make_gather(n_rows, d, batch):
    n_tc_full = n_rows // TILE
    edge_lo = n_tc_full * TILE
    edge_n = n_rows - edge_lo
    tc_base, tc_rem = divmod(n_tc_full, NUM_WORKERS)
    assert (tc_base * TILE) % CHUNK == 0 and (tc_rem * TILE) % CHUNK == 0
    out_rows = batch + NUM_WORKERS
    n_idx_steps = batch // IDX_CHUNK
    n_groups = IDX_CHUNK // LANES
    trash = batch + LANES - 1
    mesh = plsc.VectorSubcoreMesh(core_axis_name="c", subcore_axis_name="s")

    @functools.partial(
        pl.kernel,
        out_type=jax.ShapeDtypeStruct((out_rows, TILE), jnp.float32),
        mesh=mesh,
        scratch_types=[
            pltpu.VMEM((IDX_CHUNK,), jnp.int32),
            pltpu.VMEM((batch + LANES,), jnp.int32),
            pltpu.VMEM((batch + LANES,), jnp.int32),
            pltpu.VMEM((d, CHUNK), jnp.float32),
            pltpu.VMEM((d, CHUNK), jnp.float32),
            pltpu.VMEM((d, TILE), jnp.float32),
            pltpu.VMEM((ROWCAP, TILE), jnp.float32),
            pltpu.VMEM((1, TILE), jnp.int32),
            pltpu.VMEM((2 * LANES,), jnp.int32),
            pltpu.VMEM((2 * LANES,), jnp.int32),
            pltpu.SemaphoreType.DMA,
            pltpu.SemaphoreType.DMA,
            pltpu.SemaphoreType.DMA,
        ],
        compiler_params=pltpu.CompilerParams(
            use_tc_tiling_on_sc=True, needs_layout_passes=False),
    )
    def gather_kernel(idx_hbm, tab_hbm, etab_hbm, out_hbm, idxbuf, hit_loc,
                      hit_pos, cbuf_a, cbuf_b, ebuf, rowbuf, posbuf, cb_loc,
                      cb_pos, sem_a, sem_b, sem_o):
        wid = lax.axis_index("s") * NUM_CORES + lax.axis_index("c")
        iota = lax.iota(jnp.int32, LANES)
        tc0 = wid * tc_base + jnp.minimum(wid, 1) * tc_rem
        tc1 = tc0 + tc_base + jnp.where(wid < 1, tc_rem, 0)
        lo = tc0 * TILE
        hi_full = tc1 * TILE
        hi = jnp.where(wid == NUM_WORKERS - 1, n_rows, hi_full)
        sentinel = jnp.full((LANES,), batch + wid, jnp.int32)

        def reset_posbuf():
            for q in range(TILE // LANES):
                posbuf[0, pl.ds(q * LANES, LANES)] = sentinel

        reset_posbuf()
        pltpu.async_copy(tab_hbm.at[:, pl.ds(pl.multiple_of(lo, CHUNK),
                                             CHUNK)], cbuf_a, sem_a)
        pltpu.async_copy(tab_hbm.at[:, pl.ds(pl.multiple_of(lo + CHUNK,
                                             CHUNK), CHUNK)], cbuf_b, sem_b)

        m = jnp.int32(0)
        for ic in range(n_idx_steps):
            pltpu.sync_copy(idx_hbm.at[pl.ds(ic * IDX_CHUNK, IDX_CHUNK)],
                            idxbuf)

            def fbody(g, m, ic=ic):
                v = idxbuf[pl.ds(g * LANES, LANES)]
                mask = (v >= lo) & (v < hi)
                mi = mask.astype(jnp.int32)
                cnt = jnp.sum(mi)

                def append(m):
                    bpos = ic * IDX_CHUNK + g * LANES + iota
                    excl = plsc.cumsum(mi) - mi
                    tgt = jnp.where(mask, m + excl, trash)
                    plsc.store_scatter(hit_loc.at[:], [tgt], v)
                    plsc.store_scatter(hit_pos.at[:], [tgt], bpos)
                    return m + cnt

                return lax.cond(cnt > 0, append, lambda m: m, m)

            m = lax.fori_loop(0, n_groups, fbody, m)

        n_hit_groups = (m + LANES - 1) // LANES

        def fire(slot):
            del slot
            pltpu.async_copy(rowbuf, out_hbm.at[posbuf.at[0]], sem_o).wait()
            reset_posbuf()
            return jnp.int32(0)

        def extract_hits(buf, span, l0, slot):
            def hbody(h, slot):
                hl = hit_loc[pl.ds(h * LANES, LANES)]
                valid = (h * LANES + iota) < m
                inm = valid & (hl >= l0) & (hl < l0 + span)
                mi = inm.astype(jnp.int32)
                cnt = jnp.sum(mi)

                def group(slot):
                    hp = hit_pos[pl.ds(h * LANES, LANES)]
                    excl = plsc.cumsum(mi) - mi
                    ctgt = jnp.where(inm, excl, 2 * LANES - 1)
                    plsc.store_scatter(cb_loc.at[:], [ctgt], hl)
                    plsc.store_scatter(cb_pos.at[:], [ctgt], hp)
                    return slot

                def kbody(k, slot):
                    ksp = jnp.full((LANES,), k, jnp.int32)
                    colv = plsc.load_gather(cb_loc.at[:], [ksp]) - l0
                    posv = plsc.load_gather(cb_pos.at[:], [ksp])
                    slotv = jnp.full((LANES,), slot, jnp.int32)
                    for q in range(d // LANES):
                        feat = iota + q * LANES
                        vals = plsc.load_gather(buf.at[:], [feat, colv])
                        plsc.store_scatter(rowbuf.at[:], [slotv, feat], vals)
                    plsc.store_scatter(posbuf.at[:],
                                       [jnp.zeros((LANES,), jnp.int32), slotv],
                                       posv)
                    slot = slot + 1
                    return lax.cond(slot == ROWCAP, fire, lambda s: s, slot)

                def run_group(slot):
                    return lax.fori_loop(0, cnt, kbody, group(slot))

                return lax.cond(cnt > 0, run_group, lambda s: s, slot)

            return lax.fori_loop(0, n_hit_groups, hbody, slot)

        def start(buf, sem, c):
            l0 = pl.multiple_of(lo + c * CHUNK, CHUNK)
            pltpu.async_copy(tab_hbm.at[:, pl.ds(l0, CHUNK)], buf, sem)

        def drain(buf, sem):
            pltpu.make_async_copy(tab_hbm.at[:, pl.ds(0, CHUNK)], buf,
                                  sem).wait()

        n_chunks = (hi_full - lo) // CHUNK
        n_pairs = n_chunks // 2

        def cbody(ci, slot):
            c_a = 2 * ci
            drain(cbuf_a, sem_a)
            slot = extract_hits(cbuf_a, CHUNK, lo + c_a * CHUNK, slot)

            def start_a(_):
                start(cbuf_a, sem_a, c_a + 2)
                return jnp.int32(0)

            lax.cond(c_a + 2 < n_chunks, start_a,
                     lambda _: jnp.int32(0), jnp.int32(0))
            drain(cbuf_b, sem_b)
            slot = extract_hits(cbuf_b, CHUNK, lo + (c_a + 1) * CHUNK, slot)

            def start_b(_):
                start(cbuf_b, sem_b, c_a + 3)
                return jnp.int32(0)

            lax.cond(c_a + 3 < n_chunks, start_b,
                     lambda _: jnp.int32(0), jnp.int32(0))
            return slot

        slot = lax.fori_loop(0, n_pairs, cbody, jnp.int32(0))

        def odd_tail(slot):
            drain(cbuf_a, sem_a)
            return extract_hits(cbuf_a, CHUNK, lo + (n_chunks - 1) * CHUNK,
                                slot)

        slot = lax.cond(n_chunks % 2 == 1, odd_tail, lambda s: s, slot)

        pltpu.sync_copy(etab_hbm.at[:], ebuf.at[:])
        slot = extract_hits(ebuf, edge_n, edge_lo, slot)

        fire(slot)

    return gather_kernel


def kernel(idx, table):
    (batch,) = idx.shape
    n_rows, d = table.shape
    n_full = (n_rows // 128) * 128
    edge_t = jnp.pad(table[n_full:].T, ((0, 0), (0, 128 - (n_rows - n_full))))
    out_pad = _make_gather(n_rows, d, batch)(
        idx.astype(jnp.int32), table.T, edge_t)
    return out_pad[:batch, :d]


# speedup vs baseline: 1.0749x; 1.0749x over previous
import functools

import jax
import jax.numpy as jnp
from jax import lax
from jax.experimental import pallas as pl
from jax.experimental.pallas import tpu as pltpu
from jax.experimental.pallas import tpu_sc as plsc

NUM_CORES = 2
NUM_SUBCORES = 16
NUM_WORKERS = NUM_CORES * NUM_SUBCORES
LANES = 16
TILE = 128
CHUNK = 512
IDX_CHUNK = 2048
ROWCAP = 128


@functools.lru_cache(maxsize=None)
def _make_gather(n_rows, d, batch):
    n_tc_full = n_rows // TILE
    edge_lo = n_tc_full * TILE
    edge_n = n_rows - edge_lo
    tc_base, tc_rem = divmod(n_tc_full, NUM_WORKERS)
    assert (tc_base * TILE) % CHUNK == 0 and (tc_rem * TILE) % CHUNK == 0
    out_rows = batch + NUM_WORKERS
    n_idx_steps = batch // IDX_CHUNK
    n_groups = IDX_CHUNK // LANES
    trash = batch + LANES - 1
    mesh = plsc.VectorSubcoreMesh(core_axis_name="c", subcore_axis_name="s")

    @functools.partial(
        pl.kernel,
        out_type=jax.ShapeDtypeStruct((out_rows, TILE), jnp.float32),
        mesh=mesh,
        scratch_types=[
            pltpu.VMEM((IDX_CHUNK,), jnp.int32),
            pltpu.VMEM((batch + LANES,), jnp.int32),
            pltpu.VMEM((batch + LANES,), jnp.int32),
            pltpu.VMEM((d, CHUNK), jnp.float32),
            pltpu.VMEM((d, CHUNK), jnp.float32),
            pltpu.VMEM((d, TILE), jnp.float32),
            pltpu.VMEM((ROWCAP, TILE), jnp.float32),
            pltpu.VMEM((1, TILE), jnp.int32),
            pltpu.VMEM((2 * LANES,), jnp.int32),
            pltpu.VMEM((2 * LANES,), jnp.int32),
            pltpu.SemaphoreType.DMA,
            pltpu.SemaphoreType.DMA,
            pltpu.SemaphoreType.DMA,
        ],
        compiler_params=pltpu.CompilerParams(
            use_tc_tiling_on_sc=True, needs_layout_passes=False),
    )
    def gather_kernel(idx_hbm, tab_hbm, etab_hbm, out_hbm, idxbuf, hit_loc,
                      hit_pos, cbuf_a, cbuf_b, ebuf, rowbuf, posbuf, cb_loc,
                      cb_pos, sem_a, sem_b, sem_o):
        wid = lax.axis_index("s") * NUM_CORES + lax.axis_index("c")
        iota = lax.iota(jnp.int32, LANES)
        tc0 = wid * tc_base + jnp.minimum(wid, 1) * tc_rem
        tc1 = tc0 + tc_base + jnp.where(wid < 1, tc_rem, 0)
        lo = tc0 * TILE
        hi_full = tc1 * TILE
        hi = jnp.where(wid == NUM_WORKERS - 1, n_rows, hi_full)
        sentinel = jnp.full((LANES,), batch + wid, jnp.int32)

        def reset_posbuf():
            for q in range(TILE // LANES):
                posbuf[0, pl.ds(q * LANES, LANES)] = sentinel

        reset_posbuf()
        pltpu.async_copy(tab_hbm.at[:, pl.ds(pl.multiple_of(lo, CHUNK),
                                             CHUNK)], cbuf_a, sem_a)
        pltpu.async_copy(tab_hbm.at[:, pl.ds(pl.multiple_of(lo + CHUNK,
                                             CHUNK), CHUNK)], cbuf_b, sem_b)

        m = jnp.int32(0)
        for ic in range(n_idx_steps):
            pltpu.sync_copy(idx_hbm.at[pl.ds(ic * IDX_CHUNK, IDX_CHUNK)],
                            idxbuf)

            def fbody(g, m, ic=ic):
                v = idxbuf[pl.ds(g * LANES, LANES)]
                bpos = ic * IDX_CHUNK + g * LANES + iota
                mask = (v >= lo) & (v < hi)
                mi = mask.astype(jnp.int32)
                excl = plsc.cumsum(mi) - mi
                tgt = jnp.where(mask, m + excl, trash)
                plsc.store_scatter(hit_loc.at[:], [tgt], v)
                plsc.store_scatter(hit_pos.at[:], [tgt], bpos)
                return m + jnp.sum(mi)

            m = lax.fori_loop(0, n_groups, fbody, m)

        n_hit_groups = (m + LANES - 1) // LANES

        def fire(slot):
            del slot
            pltpu.async_copy(rowbuf, out_hbm.at[posbuf.at[0]], sem_o).wait()
            reset_posbuf()
            return jnp.int32(0)

        def extract_hits(buf, span, l0, slot):
            def hbody(h, slot):
                hl = hit_loc[pl.ds(h * LANES, LANES)]
                hp = hit_pos[pl.ds(h * LANES, LANES)]
                valid = (h * LANES + iota) < m
                inm = valid & (hl >= l0) & (hl < l0 + span)
                mi = inm.astype(jnp.int32)
                cnt = jnp.sum(mi)

                def group(slot):
                    excl = plsc.cumsum(mi) - mi
                    ctgt = jnp.where(inm, excl, 2 * LANES - 1)
                    plsc.store_scatter(cb_loc.at[:], [ctgt], hl)
                    plsc.store_scatter(cb_pos.at[:], [ctgt], hp)
                    return slot

                def kbody(k, slot):
                    ksp = jnp.full((LANES,), k, jnp.int32)
                    colv = plsc.load_gather(cb_loc.at[:], [ksp]) - l0
                    posv = plsc.load_gather(cb_pos.at[:], [ksp])
                    slotv = jnp.full((LANES,), slot, jnp.int32)
                    for q in range(d // LANES):
                        feat = iota + q * LANES
                        vals = plsc.load_gather(buf.at[:], [feat, colv])
                        plsc.store_scatter(rowbuf.at[:], [slotv, feat], vals)
                    plsc.store_scatter(posbuf.at[:],
                                       [jnp.zeros((LANES,), jnp.int32), slotv],
                                       posv)
                    slot = slot + 1
                    return lax.cond(slot == ROWCAP, fire, lambda s: s, slot)

                def run_group(slot):
                    return lax.fori_loop(0, cnt, kbody, group(slot))

                return lax.cond(cnt > 0, run_group, lambda s: s, slot)

            return lax.fori_loop(0, n_hit_groups, hbody, slot)

        def start(buf, sem, c):
            l0 = pl.multiple_of(lo + c * CHUNK, CHUNK)
            pltpu.async_copy(tab_hbm.at[:, pl.ds(l0, CHUNK)], buf, sem)

        def drain(buf, sem):
            pltpu.make_async_copy(tab_hbm.at[:, pl.ds(0, CHUNK)], buf,
                                  sem).wait()

        n_chunks = (hi_full - lo) // CHUNK
        n_pairs = n_chunks // 2

        def cbody(ci, slot):
            c_a = 2 * ci
            drain(cbuf_a, sem_a)
            slot = extract_hits(cbuf_a, CHUNK, lo + c_a * CHUNK, slot)

            def start_a(_):
                start(cbuf_a, sem_a, c_a + 2)
                return jnp.int32(0)

            lax.cond(c_a + 2 < n_chunks, start_a,
                     lambda _: jnp.int32(0), jnp.int32(0))
            drain(cbuf_b, sem_b)
            slot = extract_hits(cbuf_b, CHUNK, lo + (c_a + 1) * CHUNK, slot)

            def start_b(_):
                start(cbuf_b, sem_b, c_a + 3)
                return jnp.int32(0)

            lax.cond(c_a + 3 < n_chunks, start_b,
                     lambda _: jnp.int32(0), jnp.int32(0))
            return slot

        slot = lax.fori_loop(0, n_pairs, cbody, jnp.int32(0))

        def odd_tail(slot):
            drain(cbuf_a, sem_a)
            return extract_hits(cbuf_a, CHUNK, lo + (n_chunks - 1) * CHUNK,
                                slot)

        slot = lax.cond(n_chunks % 2 == 1, odd_tail, lambda s: s, slot)

        pltpu.sync_copy(etab_hbm.at[:], ebuf.at[:])
        slot = extract_hits(ebuf, edge_n, edge_lo, slot)

        fire(slot)

    return gather_kernel


def kernel(idx, table):
    (batch,) = idx.shape
    n_rows, d = table.shape
    n_full = (n_rows // 128) * 128
    edge_t = jnp.pad(table[n_full:].T, ((0, 0), (0, 128 - (n_rows - n_full))))
    out_pad = _make_gather(n_rows, d, batch)(
        idx.astype(jnp.int32), table.T, edge_t)
    return out_pad[:batch, :d]


# speedup vs baseline: 1.1138x; 1.0362x over previous
import functools

import jax
import jax.numpy as jnp
from jax import lax
from jax.experimental import pallas as pl
from jax.experimental.pallas import tpu as pltpu
from jax.experimental.pallas import tpu_sc as plsc

NUM_CORES = 2
NUM_SUBCORES = 16
NUM_WORKERS = NUM_CORES * NUM_SUBCORES
LANES = 16
TILE = 128
CHUNK = 512
IDX_CHUNK = 2048
ROWCAP = 128


@functools.lru_cache(maxsize=None)
def _make_gather(n_rows, d, batch):
    n_tc_full = n_rows // TILE
    edge_lo = n_tc_full * TILE
    edge_n = n_rows - edge_lo
    tc_base, tc_rem = divmod(n_tc_full, NUM_WORKERS)
    assert (tc_base * TILE) % CHUNK == 0 and (tc_rem * TILE) % CHUNK == 0
    out_rows = batch + NUM_WORKERS
    n_idx_steps = batch // IDX_CHUNK
    n_groups = IDX_CHUNK // LANES
    trash = batch + LANES - 1
    mesh = plsc.VectorSubcoreMesh(core_axis_name="c", subcore_axis_name="s")

    @functools.partial(
        pl.kernel,
        out_type=jax.ShapeDtypeStruct((out_rows, TILE), jnp.float32),
        mesh=mesh,
        scratch_types=[
            pltpu.VMEM((2, IDX_CHUNK), jnp.int32),
            pltpu.VMEM((batch + LANES,), jnp.int32),
            pltpu.VMEM((batch + LANES,), jnp.int32),
            pltpu.VMEM((d, CHUNK), jnp.float32),
            pltpu.VMEM((d, CHUNK), jnp.float32),
            pltpu.VMEM((d, TILE), jnp.float32),
            pltpu.VMEM((ROWCAP, TILE), jnp.float32),
            pltpu.VMEM((1, TILE), jnp.int32),
            pltpu.VMEM((2 * LANES,), jnp.int32),
            pltpu.VMEM((2 * LANES,), jnp.int32),
            pltpu.SemaphoreType.DMA,
            pltpu.SemaphoreType.DMA,
            pltpu.SemaphoreType.DMA,
        ],
        compiler_params=pltpu.CompilerParams(
            use_tc_tiling_on_sc=True, needs_layout_passes=False),
    )
    def gather_kernel(idx_hbm, tab_hbm, etab_hbm, out_hbm, idxbuf, hit_loc,
                      hit_pos, cbuf_a, cbuf_b, ebuf, rowbuf, posbuf, cb_loc,
                      cb_pos, sem_a, sem_b, sem_o):
        wid = lax.axis_index("s") * NUM_CORES + lax.axis_index("c")
        iota = lax.iota(jnp.int32, LANES)
        tc0 = wid * tc_base + jnp.minimum(wid, 1) * tc_rem
        tc1 = tc0 + tc_base + jnp.where(wid < 1, tc_rem, 0)
        lo = tc0 * TILE
        hi_full = tc1 * TILE
        hi = jnp.where(wid == NUM_WORKERS - 1, n_rows, hi_full)
        sentinel = jnp.full((LANES,), batch + wid, jnp.int32)

        def reset_posbuf():
            for q in range(TILE // LANES):
                posbuf[0, pl.ds(q * LANES, LANES)] = sentinel

        reset_posbuf()
        pltpu.async_copy(tab_hbm.at[:, pl.ds(pl.multiple_of(lo, CHUNK),
                                             CHUNK)], cbuf_a, sem_a)
        pltpu.async_copy(tab_hbm.at[:, pl.ds(pl.multiple_of(lo + CHUNK,
                                             CHUNK), CHUNK)], cbuf_b, sem_b)

        m = jnp.int32(0)
        pltpu.async_copy(idx_hbm.at[pl.ds(0, IDX_CHUNK)], idxbuf.at[0],
                         sem_o)
        for ic in range(n_idx_steps):
            cur = ic % 2
            if ic + 1 < n_idx_steps:
                pltpu.async_copy(
                    idx_hbm.at[pl.ds((ic + 1) * IDX_CHUNK, IDX_CHUNK)],
                    idxbuf.at[1 - cur], sem_o)
            pltpu.make_async_copy(idx_hbm.at[pl.ds(0, IDX_CHUNK)],
                                  idxbuf.at[cur], sem_o).wait()

            def fbody(g, m, ic=ic, cur=cur):
                b0 = g * (2 * LANES)
                v1 = idxbuf[cur, pl.ds(b0, LANES)]
                v2 = idxbuf[cur, pl.ds(b0 + LANES, LANES)]
                mask1 = (v1 >= lo) & (v1 < hi)
                mask2 = (v2 >= lo) & (v2 < hi)
                mi1 = mask1.astype(jnp.int32)
                mi2 = mask2.astype(jnp.int32)
                excl1 = plsc.cumsum(mi1) - mi1
                excl2 = plsc.cumsum(mi2) - mi2
                cnt1 = jnp.sum(mi1)
                cnt2 = jnp.sum(mi2)
                bpos1 = ic * IDX_CHUNK + b0 + iota
                tgt1 = jnp.where(mask1, m + excl1, trash)
                m1 = m + cnt1
                tgt2 = jnp.where(mask2, m1 + excl2, trash)
                plsc.store_scatter(hit_loc.at[:], [tgt1], v1)
                plsc.store_scatter(hit_pos.at[:], [tgt1], bpos1)
                plsc.store_scatter(hit_loc.at[:], [tgt2], v2)
                plsc.store_scatter(hit_pos.at[:], [tgt2], bpos1 + LANES)
                return m1 + cnt2

            m = lax.fori_loop(0, n_groups // 2, fbody, m)

        n_hit_groups = (m + LANES - 1) // LANES

        def fire(slot):
            del slot
            pltpu.async_copy(rowbuf, out_hbm.at[posbuf.at[0]], sem_o).wait()
            reset_posbuf()
            return jnp.int32(0)

        def extract_hits(buf, span, l0, slot):
            def hbody(h, slot):
                hl = hit_loc[pl.ds(h * LANES, LANES)]
                hp = hit_pos[pl.ds(h * LANES, LANES)]
                valid = (h * LANES + iota) < m
                inm = valid & (hl >= l0) & (hl < l0 + span)
                mi = inm.astype(jnp.int32)
                cnt = jnp.sum(mi)

                def group(slot):
                    excl = plsc.cumsum(mi) - mi
                    ctgt = jnp.where(inm, excl, 2 * LANES - 1)
                    plsc.store_scatter(cb_loc.at[:], [ctgt], hl)
                    plsc.store_scatter(cb_pos.at[:], [ctgt], hp)
                    return slot

                def kbody(k, slot):
                    ksp = jnp.full((LANES,), k, jnp.int32)
                    colv = plsc.load_gather(cb_loc.at[:], [ksp]) - l0
                    posv = plsc.load_gather(cb_pos.at[:], [ksp])
                    slotv = jnp.full((LANES,), slot, jnp.int32)
                    for q in range(d // LANES):
                        feat = iota + q * LANES
                        vals = plsc.load_gather(buf.at[:], [feat, colv])
                        plsc.store_scatter(rowbuf.at[:], [slotv, feat], vals)
                    plsc.store_scatter(posbuf.at[:],
                                       [jnp.zeros((LANES,), jnp.int32), slotv],
                                       posv)
                    slot = slot + 1
                    return lax.cond(slot == ROWCAP, fire, lambda s: s, slot)

                def run_group(slot):
                    return lax.fori_loop(0, cnt, kbody, group(slot))

                return lax.cond(cnt > 0, run_group, lambda s: s, slot)

            return lax.fori_loop(0, n_hit_groups, hbody, slot)

        def start(buf, sem, c):
            l0 = pl.multiple_of(lo + c * CHUNK, CHUNK)
            pltpu.async_copy(tab_hbm.at[:, pl.ds(l0, CHUNK)], buf, sem)

        def drain(buf, sem):
            pltpu.make_async_copy(tab_hbm.at[:, pl.ds(0, CHUNK)], buf,
                                  sem).wait()

        n_chunks = (hi_full - lo) // CHUNK
        n_pairs = n_chunks // 2

        def cbody(ci, slot):
            c_a = 2 * ci
            drain(cbuf_a, sem_a)
            slot = extract_hits(cbuf_a, CHUNK, lo + c_a * CHUNK, slot)

            def start_a(_):
                start(cbuf_a, sem_a, c_a + 2)
                return jnp.int32(0)

            lax.cond(c_a + 2 < n_chunks, start_a,
                     lambda _: jnp.int32(0), jnp.int32(0))
            drain(cbuf_b, sem_b)
            slot = extract_hits(cbuf_b, CHUNK, lo + (c_a + 1) * CHUNK, slot)

            def start_b(_):
                start(cbuf_b, sem_b, c_a + 3)
                return jnp.int32(0)

            lax.cond(c_a + 3 < n_chunks, start_b,
                     lambda _: jnp.int32(0), jnp.int32(0))
            return slot

        slot = lax.fori_loop(0, n_pairs, cbody, jnp.int32(0))

        def odd_tail(slot):
            drain(cbuf_a, sem_a)
            return extract_hits(cbuf_a, CHUNK, lo + (n_chunks - 1) * CHUNK,
                                slot)

        slot = lax.cond(n_chunks % 2 == 1, odd_tail, lambda s: s, slot)

        pltpu.sync_copy(etab_hbm.at[:], ebuf.at[:])
        slot = extract_hits(ebuf, edge_n, edge_lo, slot)

        fire(slot)

    return gather_kernel


def kernel(idx, table):
    (batch,) = idx.shape
    n_rows, d = table.shape
    n_full = (n_rows // 128) * 128
    edge_t = jnp.pad(table[n_full:].T, ((0, 0), (0, 128 - (n_rows - n_full))))
    out_pad = _make_gather(n_rows, d, batch)(
        idx.astype(jnp.int32), table.T, edge_t)
    return out_pad[:batch, :d]


# speedup vs baseline: 1.1408x; 1.0242x over previous
import functools

import jax
import jax.numpy as jnp
from jax import lax
from jax.experimental import pallas as pl
from jax.experimental.pallas import tpu as pltpu
from jax.experimental.pallas import tpu_sc as plsc

NUM_CORES = 2
NUM_SUBCORES = 16
NUM_WORKERS = NUM_CORES * NUM_SUBCORES
LANES = 16
TILE = 128
CHUNK = 512
IDX_CHUNK = 2048
ROWCAP = 128


@functools.lru_cache(maxsize=None)
def _make_gather(n_rows, d, batch):
    n_tc_full = n_rows // TILE
    edge_lo = n_tc_full * TILE
    edge_n = n_rows - edge_lo
    tc_base, tc_rem = divmod(n_tc_full, NUM_WORKERS)
    assert (tc_base * TILE) % CHUNK == 0 and (tc_rem * TILE) % CHUNK == 0
    out_rows = batch + NUM_WORKERS
    n_idx_steps = batch // IDX_CHUNK
    n_groups = IDX_CHUNK // LANES
    trash = batch + LANES - 1
    mesh = plsc.VectorSubcoreMesh(core_axis_name="c", subcore_axis_name="s")

    @functools.partial(
        pl.kernel,
        out_type=jax.ShapeDtypeStruct((out_rows, TILE), jnp.float32),
        mesh=mesh,
        scratch_types=[
            pltpu.VMEM((2, IDX_CHUNK), jnp.int32),
            pltpu.VMEM((batch + LANES,), jnp.int32),
            pltpu.VMEM((batch + LANES,), jnp.int32),
            pltpu.VMEM((d, CHUNK), jnp.float32),
            pltpu.VMEM((d, CHUNK), jnp.float32),
            pltpu.VMEM((d, TILE), jnp.float32),
            pltpu.VMEM((ROWCAP, TILE), jnp.float32),
            pltpu.VMEM((1, TILE), jnp.int32),
            pltpu.VMEM((2 * LANES,), jnp.int32),
            pltpu.VMEM((2 * LANES,), jnp.int32),
            pltpu.SemaphoreType.DMA,
            pltpu.SemaphoreType.DMA,
            pltpu.SemaphoreType.DMA,
        ],
        compiler_params=pltpu.CompilerParams(
            use_tc_tiling_on_sc=True, needs_layout_passes=False),
    )
    def gather_kernel(idx_hbm, tab_hbm, etab_hbm, out_hbm, idxbuf, hit_loc,
                      hit_pos, cbuf_a, cbuf_b, ebuf, rowbuf, posbuf, cb_loc,
                      cb_pos, sem_a, sem_b, sem_o):
        wid = lax.axis_index("s") * NUM_CORES + lax.axis_index("c")
        iota = lax.iota(jnp.int32, LANES)
        tc0 = wid * tc_base + jnp.minimum(wid, 1) * tc_rem
        tc1 = tc0 + tc_base + jnp.where(wid < 1, tc_rem, 0)
        lo = tc0 * TILE
        hi_full = tc1 * TILE
        hi = jnp.where(wid == NUM_WORKERS - 1, n_rows, hi_full)
        sentinel = jnp.full((LANES,), batch + wid, jnp.int32)

        def reset_posbuf():
            for q in range(TILE // LANES):
                posbuf[0, pl.ds(q * LANES, LANES)] = sentinel

        reset_posbuf()
        pltpu.async_copy(tab_hbm.at[:, pl.ds(pl.multiple_of(lo, CHUNK),
                                             CHUNK)], cbuf_a, sem_a)
        pltpu.async_copy(tab_hbm.at[:, pl.ds(pl.multiple_of(lo + CHUNK,
                                             CHUNK), CHUNK)], cbuf_b, sem_b)

        m = jnp.int32(0)
        pltpu.async_copy(idx_hbm.at[pl.ds(0, IDX_CHUNK)], idxbuf.at[0],
                         sem_o)
        for ic in range(n_idx_steps):
            cur = ic % 2
            if ic + 1 < n_idx_steps:
                pltpu.async_copy(
                    idx_hbm.at[pl.ds((ic + 1) * IDX_CHUNK, IDX_CHUNK)],
                    idxbuf.at[1 - cur], sem_o)
            pltpu.make_async_copy(idx_hbm.at[pl.ds(0, IDX_CHUNK)],
                                  idxbuf.at[cur], sem_o).wait()

            def fbody(g, m, ic=ic, cur=cur):
                b0 = g * (2 * LANES)
                v1 = idxbuf[cur, pl.ds(b0, LANES)]
                v2 = idxbuf[cur, pl.ds(b0 + LANES, LANES)]
                mask1 = (v1 >= lo) & (v1 < hi)
                mask2 = (v2 >= lo) & (v2 < hi)
                mi1 = mask1.astype(jnp.int32)
                mi2 = mask2.astype(jnp.int32)
                excl1 = plsc.cumsum(mi1) - mi1
                excl2 = plsc.cumsum(mi2) - mi2
                cnt1 = jnp.sum(mi1)
                cnt2 = jnp.sum(mi2)
                bpos1 = ic * IDX_CHUNK + b0 + iota
                tgt1 = jnp.where(mask1, m + excl1, trash)
                m1 = m + cnt1
                tgt2 = jnp.where(mask2, m1 + excl2, trash)
                plsc.store_scatter(hit_loc.at[:], [tgt1], v1)
                plsc.store_scatter(hit_pos.at[:], [tgt1], bpos1)
                plsc.store_scatter(hit_loc.at[:], [tgt2], v2)
                plsc.store_scatter(hit_pos.at[:], [tgt2], bpos1 + LANES)
                return m1 + cnt2

            m = lax.fori_loop(0, n_groups // 2, fbody, m)

        def fire(slot):
            del slot
            pltpu.async_copy(rowbuf, out_hbm.at[posbuf.at[0]], sem_o).wait()
            reset_posbuf()
            return jnp.int32(0)

        def extract_hits(buf, span, l0, slot):
            def one_group(hl, hp, inm, mi, cnt, slot):
                def group(slot):
                    excl = plsc.cumsum(mi) - mi
                    ctgt = jnp.where(inm, excl, 2 * LANES - 1)
                    plsc.store_scatter(cb_loc.at[:], [ctgt], hl)
                    plsc.store_scatter(cb_pos.at[:], [ctgt], hp)
                    return slot

                def kbody(k, slot):
                    ksp = jnp.full((LANES,), k, jnp.int32)
                    colv = plsc.load_gather(cb_loc.at[:], [ksp]) - l0
                    posv = plsc.load_gather(cb_pos.at[:], [ksp])
                    slotv = jnp.full((LANES,), slot, jnp.int32)
                    for q in range(d // LANES):
                        feat = iota + q * LANES
                        vals = plsc.load_gather(buf.at[:], [feat, colv])
                        plsc.store_scatter(rowbuf.at[:], [slotv, feat], vals)
                    plsc.store_scatter(posbuf.at[:],
                                       [jnp.zeros((LANES,), jnp.int32), slotv],
                                       posv)
                    slot = slot + 1
                    return lax.cond(slot == ROWCAP, fire, lambda s: s, slot)

                def run_group(slot):
                    return lax.fori_loop(0, cnt, kbody, group(slot))

                return lax.cond(cnt > 0, run_group, lambda s: s, slot)

            def hbody(h, slot):
                b0 = h * (2 * LANES)
                hl1 = hit_loc[pl.ds(b0, LANES)]
                hl2 = hit_loc[pl.ds(b0 + LANES, LANES)]
                hp1 = hit_pos[pl.ds(b0, LANES)]
                hp2 = hit_pos[pl.ds(b0 + LANES, LANES)]
                valid1 = (b0 + iota) < m
                valid2 = (b0 + LANES + iota) < m
                inm1 = valid1 & (hl1 >= l0) & (hl1 < l0 + span)
                inm2 = valid2 & (hl2 >= l0) & (hl2 < l0 + span)
                mi1 = inm1.astype(jnp.int32)
                mi2 = inm2.astype(jnp.int32)
                cnt1 = jnp.sum(mi1)
                cnt2 = jnp.sum(mi2)
                slot = one_group(hl1, hp1, inm1, mi1, cnt1, slot)
                return one_group(hl2, hp2, inm2, mi2, cnt2, slot)

            n_hg2 = (m + 2 * LANES - 1) // (2 * LANES)
            return lax.fori_loop(0, n_hg2, hbody, slot)

        def start(buf, sem, c):
            l0 = pl.multiple_of(lo + c * CHUNK, CHUNK)
            pltpu.async_copy(tab_hbm.at[:, pl.ds(l0, CHUNK)], buf, sem)

        def drain(buf, sem):
            pltpu.make_async_copy(tab_hbm.at[:, pl.ds(0, CHUNK)], buf,
                                  sem).wait()

        n_chunks = (hi_full - lo) // CHUNK
        n_pairs = n_chunks // 2

        def cbody(ci, slot):
            c_a = 2 * ci
            drain(cbuf_a, sem_a)
            slot = extract_hits(cbuf_a, CHUNK, lo + c_a * CHUNK, slot)

            def start_a(_):
                start(cbuf_a, sem_a, c_a + 2)
                return jnp.int32(0)

            lax.cond(c_a + 2 < n_chunks, start_a,
                     lambda _: jnp.int32(0), jnp.int32(0))
            drain(cbuf_b, sem_b)
            slot = extract_hits(cbuf_b, CHUNK, lo + (c_a + 1) * CHUNK, slot)

            def start_b(_):
                start(cbuf_b, sem_b, c_a + 3)
                return jnp.int32(0)

            lax.cond(c_a + 3 < n_chunks, start_b,
                     lambda _: jnp.int32(0), jnp.int32(0))
            return slot

        slot = lax.fori_loop(0, n_pairs, cbody, jnp.int32(0))

        def odd_tail(slot):
            drain(cbuf_a, sem_a)
            return extract_hits(cbuf_a, CHUNK, lo + (n_chunks - 1) * CHUNK,
                                slot)

        slot = lax.cond(n_chunks % 2 == 1, odd_tail, lambda s: s, slot)

        pltpu.sync_copy(etab_hbm.at[:], ebuf.at[:])
        slot = extract_hits(ebuf, edge_n, edge_lo, slot)

        fire(slot)

    return gather_kernel


def kernel(idx, table):
    (batch,) = idx.shape
    n_rows, d = table.shape
    n_full = (n_rows // 128) * 128
    edge_t = jnp.pad(table[n_full:].T, ((0, 0), (0, 128 - (n_rows - n_full))))
    out_pad = _make_gather(n_rows, d, batch)(
        idx.astype(jnp.int32), table.T, edge_t)
    return out_pad[:batch, :d]


# speedup vs baseline: 1.1665x; 1.0226x over previous
import functools

import jax
import jax.numpy as jnp
from jax import lax
from jax.experimental import pallas as pl
from jax.experimental.pallas import tpu as pltpu
from jax.experimental.pallas import tpu_sc as plsc

NUM_CORES = 2
NUM_SUBCORES = 16
NUM_WORKERS = NUM_CORES * NUM_SUBCORES
LANES = 16
TILE = 128
CHUNK = 512
IDX_CHUNK = 2048
ROWCAP = 128


@functools.lru_cache(maxsize=None)
def _make_gather(n_rows, d, batch):
    n_tc_full = n_rows // TILE
    edge_lo = n_tc_full * TILE
    edge_n = n_rows - edge_lo
    tc_base, tc_rem = divmod(n_tc_full, NUM_WORKERS)
    assert (tc_base * TILE) % CHUNK == 0 and (tc_rem * TILE) % CHUNK == 0
    out_rows = batch + NUM_WORKERS
    n_idx_steps = batch // IDX_CHUNK
    n_groups = IDX_CHUNK // LANES
    trash = batch + LANES - 1
    mesh = plsc.VectorSubcoreMesh(core_axis_name="c", subcore_axis_name="s")

    @functools.partial(
        pl.kernel,
        out_type=jax.ShapeDtypeStruct((out_rows, TILE), jnp.float32),
        mesh=mesh,
        scratch_types=[
            pltpu.VMEM((2, IDX_CHUNK), jnp.int32),
            pltpu.VMEM((batch + LANES,), jnp.int32),
            pltpu.VMEM((batch + LANES,), jnp.int32),
            pltpu.VMEM((d, CHUNK), jnp.float32),
            pltpu.VMEM((d, CHUNK), jnp.float32),
            pltpu.VMEM((d, TILE), jnp.float32),
            pltpu.VMEM((ROWCAP, TILE), jnp.float32),
            pltpu.VMEM((1, TILE), jnp.int32),
            pltpu.VMEM((2 * LANES,), jnp.int32),
            pltpu.VMEM((2 * LANES,), jnp.int32),
            pltpu.SemaphoreType.DMA,
            pltpu.SemaphoreType.DMA,
            pltpu.SemaphoreType.DMA,
        ],
        compiler_params=pltpu.CompilerParams(
            use_tc_tiling_on_sc=True, needs_layout_passes=False),
    )
    def gather_kernel(idx_hbm, tab_hbm, etab_hbm, out_hbm, idxbuf, hit_loc,
                      hit_pos, cbuf_a, cbuf_b, ebuf, rowbuf, posbuf, cb_loc,
                      cb_pos, sem_a, sem_b, sem_o):
        wid = lax.axis_index("s") * NUM_CORES + lax.axis_index("c")
        iota = lax.iota(jnp.int32, LANES)
        tc0 = wid * tc_base + jnp.minimum(wid, 1) * tc_rem
        tc1 = tc0 + tc_base + jnp.where(wid < 1, tc_rem, 0)
        lo = tc0 * TILE
        hi_full = tc1 * TILE
        hi = jnp.where(wid == NUM_WORKERS - 1, n_rows, hi_full)
        sentinel = jnp.full((LANES,), batch + wid, jnp.int32)

        def reset_posbuf():
            for q in range(TILE // LANES):
                posbuf[0, pl.ds(q * LANES, LANES)] = sentinel

        reset_posbuf()
        pltpu.async_copy(tab_hbm.at[:, pl.ds(pl.multiple_of(lo, CHUNK),
                                             CHUNK)], cbuf_a, sem_a)
        pltpu.async_copy(tab_hbm.at[:, pl.ds(pl.multiple_of(lo + CHUNK,
                                             CHUNK), CHUNK)], cbuf_b, sem_b)

        m = jnp.int32(0)
        pltpu.async_copy(idx_hbm.at[pl.ds(0, IDX_CHUNK)], idxbuf.at[0],
                         sem_o)
        for ic in range(n_idx_steps):
            cur = ic % 2
            if ic + 1 < n_idx_steps:
                pltpu.async_copy(
                    idx_hbm.at[pl.ds((ic + 1) * IDX_CHUNK, IDX_CHUNK)],
                    idxbuf.at[1 - cur], sem_o)
            pltpu.make_async_copy(idx_hbm.at[pl.ds(0, IDX_CHUNK)],
                                  idxbuf.at[cur], sem_o).wait()

            def fbody(g, m, ic=ic, cur=cur):
                b0 = g * (4 * LANES)
                vs = [idxbuf[cur, pl.ds(b0 + u * LANES, LANES)]
                      for u in range(4)]
                masks = [(v >= lo) & (v < hi) for v in vs]
                mis = [mk.astype(jnp.int32) for mk in masks]
                excls = [plsc.cumsum(mi) - mi for mi in mis]
                cnts = [jnp.sum(mi) for mi in mis]
                for u in range(4):
                    bpos = ic * IDX_CHUNK + b0 + u * LANES + iota
                    tgt = jnp.where(masks[u], m + excls[u], trash)
                    plsc.store_scatter(hit_loc.at[:], [tgt], vs[u])
                    plsc.store_scatter(hit_pos.at[:], [tgt], bpos)
                    m = m + cnts[u]
                return m

            m = lax.fori_loop(0, n_groups // 4, fbody, m)

        def fire(slot):
            del slot
            pltpu.async_copy(rowbuf, out_hbm.at[posbuf.at[0]], sem_o).wait()
            reset_posbuf()
            return jnp.int32(0)

        def extract_hits(buf, span, l0, slot):
            def one_group(hl, hp, inm, mi, cnt, slot):
                def group(slot):
                    excl = plsc.cumsum(mi) - mi
                    ctgt = jnp.where(inm, excl, 2 * LANES - 1)
                    plsc.store_scatter(cb_loc.at[:], [ctgt], hl)
                    plsc.store_scatter(cb_pos.at[:], [ctgt], hp)
                    return slot

                def kbody(k, slot):
                    ksp = jnp.full((LANES,), k, jnp.int32)
                    colv = plsc.load_gather(cb_loc.at[:], [ksp]) - l0
                    posv = plsc.load_gather(cb_pos.at[:], [ksp])
                    slotv = jnp.full((LANES,), slot, jnp.int32)
                    for q in range(d // LANES):
                        feat = iota + q * LANES
                        vals = plsc.load_gather(buf.at[:], [feat, colv])
                        plsc.store_scatter(rowbuf.at[:], [slotv, feat], vals)
                    plsc.store_scatter(posbuf.at[:],
                                       [jnp.zeros((LANES,), jnp.int32), slotv],
                                       posv)
                    slot = slot + 1
                    return lax.cond(slot == ROWCAP, fire, lambda s: s, slot)

                def run_group(slot):
                    return lax.fori_loop(0, cnt, kbody, group(slot))

                return lax.cond(cnt > 0, run_group, lambda s: s, slot)

            def hbody(h, slot):
                b0 = h * (2 * LANES)
                hl1 = hit_loc[pl.ds(b0, LANES)]
                hl2 = hit_loc[pl.ds(b0 + LANES, LANES)]
                hp1 = hit_pos[pl.ds(b0, LANES)]
                hp2 = hit_pos[pl.ds(b0 + LANES, LANES)]
                valid1 = (b0 + iota) < m
                valid2 = (b0 + LANES + iota) < m
                inm1 = valid1 & (hl1 >= l0) & (hl1 < l0 + span)
                inm2 = valid2 & (hl2 >= l0) & (hl2 < l0 + span)
                mi1 = inm1.astype(jnp.int32)
                mi2 = inm2.astype(jnp.int32)
                cnt1 = jnp.sum(mi1)
                cnt2 = jnp.sum(mi2)
                slot = one_group(hl1, hp1, inm1, mi1, cnt1, slot)
                return one_group(hl2, hp2, inm2, mi2, cnt2, slot)

            n_hg2 = (m + 2 * LANES - 1) // (2 * LANES)
            return lax.fori_loop(0, n_hg2, hbody, slot)

        def start(buf, sem, c):
            l0 = pl.multiple_of(lo + c * CHUNK, CHUNK)
            pltpu.async_copy(tab_hbm.at[:, pl.ds(l0, CHUNK)], buf, sem)

        def drain(buf, sem):
            pltpu.make_async_copy(tab_hbm.at[:, pl.ds(0, CHUNK)], buf,
                                  sem).wait()

        n_chunks = (hi_full - lo) // CHUNK
        n_pairs = n_chunks // 2

        def cbody(ci, slot):
            c_a = 2 * ci
            drain(cbuf_a, sem_a)
            slot = extract_hits(cbuf_a, CHUNK, lo + c_a * CHUNK, slot)

            def start_a(_):
                start(cbuf_a, sem_a, c_a + 2)
                return jnp.int32(0)

            lax.cond(c_a + 2 < n_chunks, start_a,
                     lambda _: jnp.int32(0), jnp.int32(0))
            drain(cbuf_b, sem_b)
            slot = extract_hits(cbuf_b, CHUNK, lo + (c_a + 1) * CHUNK, slot)

            def start_b(_):
                start(cbuf_b, sem_b, c_a + 3)
                return jnp.int32(0)

            lax.cond(c_a + 3 < n_chunks, start_b,
                     lambda _: jnp.int32(0), jnp.int32(0))
            return slot

        slot = lax.fori_loop(0, n_pairs, cbody, jnp.int32(0))

        def odd_tail(slot):
            drain(cbuf_a, sem_a)
            return extract_hits(cbuf_a, CHUNK, lo + (n_chunks - 1) * CHUNK,
                                slot)

        slot = lax.cond(n_chunks % 2 == 1, odd_tail, lambda s: s, slot)

        pltpu.sync_copy(etab_hbm.at[:], ebuf.at[:])
        slot = extract_hits(ebuf, edge_n, edge_lo, slot)

        fire(slot)

    return gather_kernel


def kernel(idx, table):
    (batch,) = idx.shape
    n_rows, d = table.shape
    n_full = (n_rows // 128) * 128
    edge_t = jnp.pad(table[n_full:].T, ((0, 0), (0, 128 - (n_rows - n_full))))
    out_pad = _make_gather(n_rows, d, batch)(
        idx.astype(jnp.int32), table.T, edge_t)
    return out_pad[:batch, :d]
